# baseline (speedup 1.0000x reference)
import functools

import jax
import jax.numpy as jnp
from jax import lax
from jax.experimental import pallas as pl
from jax.experimental.pallas import tpu as pltpu
from jax.experimental.pallas import tpu_sc as plsc

N_ROWS = 128
N_COLS = 32768
NC = 2
NS = 16
LANES = 16
TROW = 8
N_TROWS = N_ROWS // TROW
HALF_COLS = N_COLS // NC
CH = 2048
N_CHUNKS = HALF_COLS // CH


def _lane_min(m):
    lane = lax.iota(jnp.int32, LANES)
    for sh in (8, 4, 2, 1):
        perm = jnp.bitwise_xor(lane, sh)
        shuf = lax.gather(
            m, perm[:, None],
            lax.GatherDimensionNumbers(
                offset_dims=(), collapsed_slice_dims=(0,),
                start_index_map=(0,)),
            slice_sizes=(1,),
            mode=lax.GatherScatterMode.PROMISE_IN_BOUNDS)
        m = jnp.minimum(m, shuf)
    return m


def _reduce_chunk(buf, acc):
    @plsc.parallel_loop(0, CH // 128, unroll=2, carry=acc)
    def body(cb, a):
        base = cb * 128
        new = []
        for r in range(TROW):
            ar = a[r]
            for l in range(128 // LANES):
                ar = jnp.minimum(ar, buf[r, pl.ds(base + l * LANES, LANES)])
            new.append(ar)
        return tuple(new)
    return body


_mesh = plsc.VectorSubcoreMesh(core_axis_name="c", subcore_axis_name="s")


@functools.partial(
    pl.kernel,
    out_type=jax.ShapeDtypeStruct((NC, N_ROWS, 128), jnp.float32),
    mesh=_mesh,
    scratch_types=[
        pltpu.VMEM((TROW, CH), jnp.float32),
        pltpu.VMEM((TROW, CH), jnp.float32),
        pltpu.VMEM((TROW, 128), jnp.float32),
        pltpu.SemaphoreType.DMA,
        pltpu.SemaphoreType.DMA,
    ],
    compiler_params=pltpu.CompilerParams(use_tc_tiling_on_sc=True),
)
def _row_min_kernel(x_hbm, out_hbm, buf0, buf1, outstage, sem0, sem1):
    c = lax.axis_index("c")
    s = lax.axis_index("s")
    row0 = s * TROW
    col0 = c * HALF_COLS

    bufs = (buf0, buf1)
    sems = (sem0, sem1)

    def start(k):
        return pltpu.async_copy(
            x_hbm.at[pl.ds(row0, TROW), pl.ds(col0 + k * CH, CH)],
            bufs[k % 2], sems[k % 2])

    copies = [None] * (N_CHUNKS + 1)
    copies[0] = start(0)

    pos_inf = jnp.full((LANES,), jnp.inf, jnp.float32)
    acc = (pos_inf,) * TROW
    for k in range(N_CHUNKS):
        if k + 1 < N_CHUNKS:
            copies[k + 1] = start(k + 1)
        copies[k].wait()
        acc = _reduce_chunk(bufs[k % 2], acc)

    for r in range(TROW):
        m = _lane_min(acc[r])
        for l in range(128 // LANES):
            outstage[r, pl.ds(l * LANES, LANES)] = m

    pltpu.sync_copy(outstage,
                    out_hbm.at[c, pl.ds(row0, TROW), pl.ds(0, 128)])


def kernel(x):
    staged = _row_min_kernel(x)
    return jnp.minimum(staged[0, :, :1], staged[1, :, :1])


# speedup vs baseline: 2.1198x; 2.1198x over previous
import functools

import jax
import jax.numpy as jnp
from jax import lax
from jax.experimental import pallas as pl
from jax.experimental.pallas import tpu as pltpu
from jax.experimental.pallas import tpu_sc as plsc

N_ROWS = 128
N_COLS = 32768
NC = 2
NS = 16
LANES = 16
TROW = 8
N_TROWS = N_ROWS // TROW
HALF_COLS = N_COLS // NC
CH = 2048
N_CHUNKS = HALF_COLS // CH


def _lane_min(m):
    lane = lax.iota(jnp.int32, LANES)
    for sh in (8, 4, 2, 1):
        perm = jnp.bitwise_xor(lane, sh)
        shuf = lax.gather(
            m, perm[:, None],
            lax.GatherDimensionNumbers(
                offset_dims=(), collapsed_slice_dims=(0,),
                start_index_map=(0,)),
            slice_sizes=(1,),
            mode=lax.GatherScatterMode.PROMISE_IN_BOUNDS)
        m = jnp.minimum(m, shuf)
    return m


def _reduce_chunk(buf, acc):
    @plsc.parallel_loop(0, CH // 128, unroll=2, carry=acc)
    def body(cb, a):
        base = cb * 128
        new = []
        for r in range(TROW):
            ar = a[r]
            for l in range(128 // LANES):
                ar = jnp.minimum(ar, buf[r, pl.ds(base + l * LANES, LANES)])
            new.append(ar)
        return tuple(new)
    return body


_mesh = plsc.VectorSubcoreMesh(core_axis_name="c", subcore_axis_name="s")


@functools.partial(
    pl.kernel,
    out_type=jax.ShapeDtypeStruct((NC, N_ROWS, 128), jnp.float32),
    mesh=_mesh,
    scratch_types=[
        pltpu.VMEM((TROW, CH), jnp.float32),
        pltpu.VMEM((TROW, CH), jnp.float32),
        pltpu.VMEM((TROW, 128), jnp.float32),
        pltpu.SemaphoreType.DMA,
        pltpu.SemaphoreType.DMA,
    ],
    compiler_params=pltpu.CompilerParams(use_tc_tiling_on_sc=True),
)
def _row_min_kernel(x_hbm, out_hbm, buf0, buf1, outstage, sem0, sem1):
    c = lax.axis_index("c")
    s = lax.axis_index("s")
    row0 = s * TROW
    col0 = c * HALF_COLS

    bufs = (buf0, buf1)
    sems = (sem0, sem1)

    def start(k):
        return pltpu.async_copy(
            x_hbm.at[pl.ds(row0, TROW), pl.ds(col0 + k * CH, CH)],
            bufs[k % 2], sems[k % 2])

    copies = [None] * (N_CHUNKS + 1)
    copies[0] = start(0)

    pos_inf = jnp.full((LANES,), jnp.inf, jnp.float32)
    acc = (pos_inf,) * TROW
    for k in range(N_CHUNKS):
        if k + 1 < N_CHUNKS:
            copies[k + 1] = start(k + 1)
        copies[k].wait()
        acc = _reduce_chunk(bufs[k % 2], acc)

    for r in range(TROW):
        m = _lane_min(acc[r])
        for l in range(128 // LANES):
            outstage[r, pl.ds(l * LANES, LANES)] = m

    pltpu.sync_copy(outstage,
                    out_hbm.at[c, pl.ds(row0, TROW), pl.ds(0, 128)])


TC_BLK = 2048


def _tc_body(x_ref, out_ref):
    j = pl.program_id(0)
    blk = x_ref[...]
    part = jnp.min(blk.reshape(N_ROWS, TC_BLK // 128, 128), axis=1)

    @pl.when(j == 0)
    def _():
        out_ref[...] = jnp.full((N_ROWS, 128), jnp.inf, jnp.float32)

    out_ref[...] = jnp.minimum(out_ref[...], part)


def _tc_row_min(x):
    return pl.pallas_call(
        _tc_body,
        grid=(N_COLS // TC_BLK,),
        in_specs=[pl.BlockSpec((N_ROWS, TC_BLK), lambda j: (0, j))],
        out_specs=pl.BlockSpec((N_ROWS, 128), lambda j: (0, 0)),
        out_shape=jax.ShapeDtypeStruct((N_ROWS, 128), jnp.float32),
    )(x)


def kernel(x):
    staged = _tc_row_min(x)
    return jnp.min(staged, axis=1, keepdims=True)


# speedup vs baseline: 2.1239x; 1.0019x over previous
import functools

import jax
import jax.numpy as jnp
from jax import lax
from jax.experimental import pallas as pl
from jax.experimental.pallas import tpu as pltpu
from jax.experimental.pallas import tpu_sc as plsc

N_ROWS = 128
N_COLS = 32768
NC = 2
NS = 16
LANES = 16
TROW = 8
N_TROWS = N_ROWS // TROW
HALF_COLS = N_COLS // NC
CH = 2048
N_CHUNKS = HALF_COLS // CH


def _lane_min(m):
    lane = lax.iota(jnp.int32, LANES)
    for sh in (8, 4, 2, 1):
        perm = jnp.bitwise_xor(lane, sh)
        shuf = lax.gather(
            m, perm[:, None],
            lax.GatherDimensionNumbers(
                offset_dims=(), collapsed_slice_dims=(0,),
                start_index_map=(0,)),
            slice_sizes=(1,),
            mode=lax.GatherScatterMode.PROMISE_IN_BOUNDS)
        m = jnp.minimum(m, shuf)
    return m


def _reduce_chunk(buf, acc):
    @plsc.parallel_loop(0, CH // 128, unroll=2, carry=acc)
    def body(cb, a):
        base = cb * 128
        new = []
        for r in range(TROW):
            ar = a[r]
            for l in range(128 // LANES):
                ar = jnp.minimum(ar, buf[r, pl.ds(base + l * LANES, LANES)])
            new.append(ar)
        return tuple(new)
    return body


_mesh = plsc.VectorSubcoreMesh(core_axis_name="c", subcore_axis_name="s")


@functools.partial(
    pl.kernel,
    out_type=jax.ShapeDtypeStruct((NC, N_ROWS, 128), jnp.float32),
    mesh=_mesh,
    scratch_types=[
        pltpu.VMEM((TROW, CH), jnp.float32),
        pltpu.VMEM((TROW, CH), jnp.float32),
        pltpu.VMEM((TROW, 128), jnp.float32),
        pltpu.SemaphoreType.DMA,
        pltpu.SemaphoreType.DMA,
    ],
    compiler_params=pltpu.CompilerParams(use_tc_tiling_on_sc=True),
)
def _row_min_kernel(x_hbm, out_hbm, buf0, buf1, outstage, sem0, sem1):
    c = lax.axis_index("c")
    s = lax.axis_index("s")
    row0 = s * TROW
    col0 = c * HALF_COLS

    bufs = (buf0, buf1)
    sems = (sem0, sem1)

    def start(k):
        return pltpu.async_copy(
            x_hbm.at[pl.ds(row0, TROW), pl.ds(col0 + k * CH, CH)],
            bufs[k % 2], sems[k % 2])

    copies = [None] * (N_CHUNKS + 1)
    copies[0] = start(0)

    pos_inf = jnp.full((LANES,), jnp.inf, jnp.float32)
    acc = (pos_inf,) * TROW
    for k in range(N_CHUNKS):
        if k + 1 < N_CHUNKS:
            copies[k + 1] = start(k + 1)
        copies[k].wait()
        acc = _reduce_chunk(bufs[k % 2], acc)

    for r in range(TROW):
        m = _lane_min(acc[r])
        for l in range(128 // LANES):
            outstage[r, pl.ds(l * LANES, LANES)] = m

    pltpu.sync_copy(outstage,
                    out_hbm.at[c, pl.ds(row0, TROW), pl.ds(0, 128)])


TC_BLK = 2048


def _tc_body(x_ref, out_ref, acc_ref):
    j = pl.program_id(0)
    blk = x_ref[...]
    part = jnp.min(blk.reshape(N_ROWS, TC_BLK // 128, 128), axis=1)

    @pl.when(j == 0)
    def _():
        acc_ref[...] = jnp.full((N_ROWS, 128), jnp.inf, jnp.float32)

    acc_ref[...] = jnp.minimum(acc_ref[...], part)

    @pl.when(j == pl.num_programs(0) - 1)
    def _():
        out_ref[...] = jnp.min(acc_ref[...], axis=1, keepdims=True)


def _tc_row_min(x):
    return pl.pallas_call(
        _tc_body,
        grid=(N_COLS // TC_BLK,),
        in_specs=[pl.BlockSpec((N_ROWS, TC_BLK), lambda j: (0, j))],
        out_specs=pl.BlockSpec((N_ROWS, 1), lambda j: (0, 0)),
        out_shape=jax.ShapeDtypeStruct((N_ROWS, 1), jnp.float32),
        scratch_shapes=[pltpu.VMEM((N_ROWS, 128), jnp.float32)],
    )(x)


def kernel(x):
    return _tc_row_min(x)


# speedup vs baseline: 3.5091x; 1.6522x over previous
import jax
import jax.numpy as jnp
from jax.experimental import pallas as pl
from jax.experimental.pallas import tpu as pltpu

N_ROWS = 128
N_COLS = 32768
TC_BLK = 16384
TC_RBLK = 128


def _tc_body(x_ref, out_ref, acc_ref):
    j = pl.program_id(1)
    blk = x_ref[...]
    part = jnp.min(blk.reshape(TC_RBLK, TC_BLK // 128, 128), axis=1)

    @pl.when(j == 0)
    def _():
        acc_ref[...] = jnp.full((TC_RBLK, 128), jnp.inf, jnp.float32)

    acc_ref[...] = jnp.minimum(acc_ref[...], part)

    @pl.when(j == pl.num_programs(1) - 1)
    def _():
        out_ref[...] = jnp.min(acc_ref[...], axis=1, keepdims=True)


def kernel(x):
    return pl.pallas_call(
        _tc_body,
        grid=(N_ROWS // TC_RBLK, N_COLS // TC_BLK),
        in_specs=[pl.BlockSpec((TC_RBLK, TC_BLK), lambda i, j: (i, j))],
        out_specs=pl.BlockSpec((TC_RBLK, 1), lambda i, j: (i, 0)),
        out_shape=jax.ShapeDtypeStruct((N_ROWS, 1), jnp.float32),
        scratch_shapes=[pltpu.VMEM((TC_RBLK, 128), jnp.float32)],
    )(x)
